# baseline (speedup 1.0000x reference)
import functools

import jax
import jax.numpy as jnp
import numpy as np
from jax import lax
from jax.experimental import pallas as pl
from jax.experimental.pallas import tpu as pltpu
from jax.experimental.pallas import tpu_sc as plsc

N_NODES = 10000
N_PAD = 10240
N_EDGES = 320000
C = 128

NC = 2
NS = 16
NW = NC * NS
CHUNK = 128
NHALF = 40
NCHUNK = 2 * NHALF
E_PAD = NW * NCHUNK * CHUNK
E_PER_W = E_PAD // NW
NBUF = 2
ROWS_PER_TILE = N_PAD // NS

_mesh = plsc.VectorSubcoreMesh(
    core_axis_name="c", subcore_axis_name="s", num_cores=NC, num_subcores=NS)
_sc_params = pltpu.CompilerParams(needs_layout_passes=False)


def _hist_body(dst_hbm, hist_out, didx, hist):
    c = lax.axis_index("c")
    s = lax.axis_index("s")
    w = s * NC + c

    def _zero(i, carry):
        hist[pl.ds(i * 16, 16)] = jnp.zeros((16,), jnp.float32)
        return carry
    lax.fori_loop(0, N_PAD // 16, _zero, 0)

    pltpu.sync_copy(dst_hbm.at[w], didx)

    ones = jnp.ones((16,), jnp.float32)

    def _acc(j, carry):
        iv = didx[pl.ds(j * 16, 16)]
        plsc.addupdate_scatter(hist, [iv], ones)
        return carry
    lax.fori_loop(0, E_PER_W // 16, _acc, 0)

    pltpu.sync_copy(hist, hist_out.at[w])


_hist_call = functools.partial(
    pl.kernel,
    out_type=jax.ShapeDtypeStruct((NW, N_PAD), jnp.float32),
    mesh=_mesh,
    scratch_types=[
        pltpu.VMEM((E_PER_W,), jnp.int32),
        pltpu.VMEM((N_PAD,), jnp.float32),
    ],
    compiler_params=_sc_params,
)(_hist_body)


def _mm_body(x_ref, hist_ref, w_ref, g_ref):
    deg = jnp.sum(hist_ref[...], axis=0) + 1.0
    s = lax.rsqrt(deg)
    xs = x_ref[...] * s[:, None]
    g_ref[...] = jnp.dot(xs, w_ref[...], preferred_element_type=jnp.float32)


_MM_R = 1024

def _mm_call(x_pad, hist, W):
    return pl.pallas_call(
        _mm_body,
        grid=(N_PAD // _MM_R,),
        in_specs=[
            pl.BlockSpec((_MM_R, C), lambda i: (i, 0)),
            pl.BlockSpec((NW, _MM_R), lambda i: (0, i)),
            pl.BlockSpec((C, C), lambda i: (0, 0)),
        ],
        out_specs=pl.BlockSpec((_MM_R, C), lambda i: (i, 0)),
        out_shape=jax.ShapeDtypeStruct((N_PAD, C), jnp.float32),
    )(x_pad, hist, W)


def _gs_body(g_hbm, src_hbm, dst_hbm, acc_out, sidx, didx, rows, acc_sh,
             sg0, sg1):
    sem_g = (sg0, sg1)
    c = lax.axis_index("c")
    s = lax.axis_index("s")
    w = s * NC + c
    t = s

    def _pair(p, carry):
        j0 = 2 * p
        j1 = 2 * p + 1
        d0 = pltpu.async_copy(g_hbm.at[sidx.at[j0]], rows.at[0], sem_g[0])
        d1 = pltpu.async_copy(g_hbm.at[sidx.at[j1]], rows.at[1], sem_g[1])
        d0.wait()
        pltpu.sync_copy(rows.at[0], acc_sh.at[didx.at[j0]], add=True)
        d1.wait()
        pltpu.sync_copy(rows.at[1], acc_sh.at[didx.at[j1]], add=True)
        return carry

    pltpu.sync_copy(src_hbm.at[w, 0], sidx)
    pltpu.sync_copy(dst_hbm.at[w, 0], didx)
    pltpu.sync_copy(g_hbm.at[pl.ds(t * ROWS_PER_TILE, ROWS_PER_TILE)],
                    acc_sh.at[pl.ds(t * ROWS_PER_TILE, ROWS_PER_TILE)])
    plsc.subcore_barrier()
    lax.fori_loop(0, NHALF // 2, _pair, 0)

    pltpu.sync_copy(src_hbm.at[w, 1], sidx)
    pltpu.sync_copy(dst_hbm.at[w, 1], didx)
    lax.fori_loop(0, NHALF // 2, _pair, 0)

    plsc.subcore_barrier()
    pltpu.sync_copy(acc_sh.at[pl.ds(t * ROWS_PER_TILE, ROWS_PER_TILE)],
                    acc_out.at[c, pl.ds(t * ROWS_PER_TILE, ROWS_PER_TILE)])


_gs_call = functools.partial(
    pl.kernel,
    out_type=jax.ShapeDtypeStruct((NC, N_PAD, C), jnp.float32),
    mesh=_mesh,
    scratch_types=[
        pltpu.VMEM((NHALF, CHUNK), jnp.int32),
        pltpu.VMEM((NHALF, CHUNK), jnp.int32),
        pltpu.VMEM((NBUF, CHUNK, C), jnp.float32),
        pltpu.VMEM_SHARED((N_PAD, C), jnp.float32),
    ] + [pltpu.SemaphoreType.DMA] * NBUF,
    compiler_params=_sc_params,
)(_gs_body)


def _fin_body(acc_ref, g_ref, hist_ref, gb_ref, b2_ref, o_ref):
    deg = jnp.sum(hist_ref[...], axis=0) + 1.0
    s = lax.rsqrt(deg)
    tot = acc_ref[0] + acc_ref[1] - g_ref[...]
    o_ref[...] = jnp.maximum(tot * s[:, None] * gb_ref[...] + b2_ref[...], 0.0)


def _fin_call(acc, g, hist, gb, b2):
    return pl.pallas_call(
        _fin_body,
        grid=(N_PAD // _MM_R,),
        in_specs=[
            pl.BlockSpec((NC, _MM_R, C), lambda i: (0, i, 0)),
            pl.BlockSpec((_MM_R, C), lambda i: (i, 0)),
            pl.BlockSpec((NW, _MM_R), lambda i: (0, i)),
            pl.BlockSpec((1, C), lambda i: (0, 0)),
            pl.BlockSpec((1, C), lambda i: (0, 0)),
        ],
        out_specs=pl.BlockSpec((_MM_R, C), lambda i: (i, 0)),
        out_shape=jax.ShapeDtypeStruct((N_PAD, C), jnp.float32),
    )(acc, g, hist, gb, b2)


def kernel(x, edge_index, W, b, gamma, beta):
    src = edge_index[0].astype(jnp.int32)
    dst = edge_index[1].astype(jnp.int32)
    trash = jnp.full((E_PAD - N_EDGES,), N_PAD - 1, dtype=jnp.int32)
    srcp = jnp.concatenate([src, trash])
    dstp = jnp.concatenate([dst, trash])
    src3 = srcp.reshape(NW, 2, NHALF, CHUNK)
    dst3 = dstp.reshape(NW, 2, NHALF, CHUNK)
    x_pad = jnp.pad(x, ((0, N_PAD - N_NODES), (0, 0)))

    hist = _hist_call(dstp.reshape(NW, E_PER_W))
    g = _mm_call(x_pad, hist, W)
    acc = _gs_call(g, src3, dst3)

    gb = (gamma / np.sqrt(1.0 + 1e-5)).reshape(1, C)
    b2 = (b * gb[0] + beta).reshape(1, C)
    out = _fin_call(acc, g, hist, gb, b2)
    return out[:N_NODES]


# speedup vs baseline: 2.6299x; 2.6299x over previous
import functools

import jax
import jax.numpy as jnp
import numpy as np
from jax import lax
from jax.experimental import pallas as pl
from jax.experimental.pallas import tpu as pltpu
from jax.experimental.pallas import tpu_sc as plsc

N_NODES = 10000
N_PAD = 10240
N_EDGES = 320000
C = 128

NC = 2
NS = 16
NW = NC * NS
CHUNK = 128
NHALF = 40
NCHUNK = 2 * NHALF
E_PAD = NW * NCHUNK * CHUNK
E_PER_W = E_PAD // NW
NBUF = 2
ROWS_PER_TILE = N_PAD // NS

_mesh = plsc.VectorSubcoreMesh(
    core_axis_name="c", subcore_axis_name="s", num_cores=NC, num_subcores=NS)
_sc_params = pltpu.CompilerParams(needs_layout_passes=False)


def _hist_body(dst_hbm, hist_out, didx, hist):
    c = lax.axis_index("c")
    s = lax.axis_index("s")
    w = s * NC + c

    def _zero(i, carry):
        hist[pl.ds(i * 16, 16)] = jnp.zeros((16,), jnp.float32)
        return carry
    lax.fori_loop(0, N_PAD // 16, _zero, 0)

    pltpu.sync_copy(dst_hbm.at[w], didx)

    ones = jnp.ones((16,), jnp.float32)

    def _acc(j, carry):
        iv = didx[pl.ds(j * 16, 16)]
        plsc.addupdate_scatter(hist, [iv], ones)
        return carry
    lax.fori_loop(0, E_PER_W // 16, _acc, 0)

    pltpu.sync_copy(hist, hist_out.at[w])


_hist_call = functools.partial(
    pl.kernel,
    out_type=jax.ShapeDtypeStruct((NW, N_PAD), jnp.float32),
    mesh=_mesh,
    scratch_types=[
        pltpu.VMEM((E_PER_W,), jnp.int32),
        pltpu.VMEM((N_PAD,), jnp.float32),
    ],
    compiler_params=_sc_params,
)(_hist_body)


def _mm_body(x_ref, hist_ref, w_ref, g_ref):
    deg = jnp.sum(hist_ref[...], axis=0) + 1.0
    s = lax.rsqrt(deg)
    xs = x_ref[...] * s[:, None]
    g_ref[...] = jnp.dot(xs, w_ref[...], preferred_element_type=jnp.float32)


_MM_R = 1024

def _mm_call(x_pad, hist, W):
    return pl.pallas_call(
        _mm_body,
        grid=(N_PAD // _MM_R,),
        in_specs=[
            pl.BlockSpec((_MM_R, C), lambda i: (i, 0)),
            pl.BlockSpec((NW, _MM_R), lambda i: (0, i)),
            pl.BlockSpec((C, C), lambda i: (0, 0)),
        ],
        out_specs=pl.BlockSpec((_MM_R, C), lambda i: (i, 0)),
        out_shape=jax.ShapeDtypeStruct((N_PAD, C), jnp.float32),
    )(x_pad, hist, W)


def _gs_body(g_hbm, src_hbm, dst_hbm, acc_out, sidx, didx, rows, acc_sh,
             sg0, sg1):
    sem_g = (sg0, sg1)
    c = lax.axis_index("c")
    s = lax.axis_index("s")
    w = s * NC + c
    t = s

    def _pair(p, carry):
        j0 = 2 * p
        j1 = 2 * p + 1
        d0 = pltpu.async_copy(g_hbm.at[sidx.at[j0]], rows.at[0], sem_g[0])
        d1 = pltpu.async_copy(g_hbm.at[sidx.at[j1]], rows.at[1], sem_g[1])
        d0.wait()
        pltpu.sync_copy(rows.at[0], acc_sh.at[didx.at[j0]], add=True)
        d1.wait()
        pltpu.sync_copy(rows.at[1], acc_sh.at[didx.at[j1]], add=True)
        return carry

    pltpu.sync_copy(src_hbm.at[w, 0], sidx)
    pltpu.sync_copy(dst_hbm.at[w, 0], didx)
    pltpu.sync_copy(g_hbm.at[pl.ds(t * ROWS_PER_TILE, ROWS_PER_TILE)],
                    acc_sh.at[pl.ds(t * ROWS_PER_TILE, ROWS_PER_TILE)])
    plsc.subcore_barrier()
    lax.fori_loop(0, NHALF // 2, _pair, 0)

    pltpu.sync_copy(src_hbm.at[w, 1], sidx)
    pltpu.sync_copy(dst_hbm.at[w, 1], didx)
    lax.fori_loop(0, NHALF // 2, _pair, 0)

    plsc.subcore_barrier()
    pltpu.sync_copy(acc_sh.at[pl.ds(t * ROWS_PER_TILE, ROWS_PER_TILE)],
                    acc_out.at[c, pl.ds(t * ROWS_PER_TILE, ROWS_PER_TILE)])


_gs_call = functools.partial(
    pl.kernel,
    out_type=jax.ShapeDtypeStruct((NC, N_PAD, C), jnp.float32),
    mesh=_mesh,
    scratch_types=[
        pltpu.VMEM((NHALF, CHUNK), jnp.int32),
        pltpu.VMEM((NHALF, CHUNK), jnp.int32),
        pltpu.VMEM((NBUF, CHUNK, C), jnp.float32),
        pltpu.VMEM_SHARED((N_PAD, C), jnp.float32),
    ] + [pltpu.SemaphoreType.DMA] * NBUF,
    compiler_params=_sc_params,
)(_gs_body)


def _fin_body(acc_ref, g_ref, hist_ref, gb_ref, b2_ref, o_ref):
    deg = jnp.sum(hist_ref[...], axis=0) + 1.0
    s = lax.rsqrt(deg)
    tot = acc_ref[0] + acc_ref[1] - g_ref[...]
    o_ref[...] = jnp.maximum(tot * s[:, None] * gb_ref[...] + b2_ref[...], 0.0)


def _fin_call(acc, g, hist, gb, b2):
    return pl.pallas_call(
        _fin_body,
        grid=(N_PAD // _MM_R,),
        in_specs=[
            pl.BlockSpec((NC, _MM_R, C), lambda i: (0, i, 0)),
            pl.BlockSpec((_MM_R, C), lambda i: (i, 0)),
            pl.BlockSpec((NW, _MM_R), lambda i: (0, i)),
            pl.BlockSpec((1, C), lambda i: (0, 0)),
            pl.BlockSpec((1, C), lambda i: (0, 0)),
        ],
        out_specs=pl.BlockSpec((_MM_R, C), lambda i: (i, 0)),
        out_shape=jax.ShapeDtypeStruct((N_PAD, C), jnp.float32),
    )(acc, g, hist, gb, b2)


def kernel(x, edge_index, W, b, gamma, beta):
    src = edge_index[0].astype(jnp.int32)
    dst = edge_index[1].astype(jnp.int32)
    trash = N_NODES + (jnp.arange(E_PAD - N_EDGES, dtype=jnp.int32)
                       % (N_PAD - N_NODES))
    srcp = jnp.concatenate([src, trash])
    dstp = jnp.concatenate([dst, trash])
    src3 = srcp.reshape(NW, 2, NHALF, CHUNK)
    dst3 = dstp.reshape(NW, 2, NHALF, CHUNK)
    x_pad = jnp.pad(x, ((0, N_PAD - N_NODES), (0, 0)))

    hist = _hist_call(dstp.reshape(NW, E_PER_W))
    g = _mm_call(x_pad, hist, W)
    acc = _gs_call(g, src3, dst3)

    gb = (gamma / np.sqrt(1.0 + 1e-5)).reshape(1, C)
    b2 = (b * gb[0] + beta).reshape(1, C)
    out = _fin_call(acc, g, hist, gb, b2)
    return out[:N_NODES]
